# baseline (speedup 1.0000x reference)
import jax
import jax.numpy as jnp
from jax.experimental import pallas as pl
from jax.experimental.pallas import tpu as pltpu

_LB = 128


def _vq_argmin_kernel(x_hbm, c_hbm, out_ref, xbuf, cbuf, sem_x, sem_c):
    x_copy = pltpu.make_async_copy(x_hbm, xbuf, sem_x)
    x_copy.start()
    c_copy = pltpu.make_async_copy(c_hbm, cbuf, sem_c)
    c_copy.start()

    c_copy.wait()
    ct2 = cbuf[...].T * -2.0
    k = ct2.shape[1]
    cnorm = 0.25 * jnp.sum(ct2 * ct2, axis=0, keepdims=True)
    x_copy.wait()
    g2 = jnp.dot(xbuf[...], ct2, preferred_element_type=jnp.float32,
                 precision=jax.lax.Precision.HIGHEST)

    m8 = cnorm[:, :_LB] + g2[:, :_LB]
    a8 = jnp.zeros(m8.shape, jnp.int32)
    for c in range(1, k // _LB):
        s_c = cnorm[:, c * _LB:(c + 1) * _LB] + g2[:, c * _LB:(c + 1) * _LB]
        lt = s_c < m8
        m8 = jnp.where(lt, s_c, m8)
        a8 = jnp.where(lt, c, a8)
    lane = jax.lax.broadcasted_iota(jnp.int32, m8.shape, 1)
    idxp = a8 * _LB + lane
    m_t = m8.T
    i_t = idxp.T
    m = jnp.min(m_t, axis=0, keepdims=True)
    idx = jnp.min(jnp.where(m_t == m, i_t, k), axis=0)
    out_ref[...] = idx


def kernel(x, centroids):
    n, d = x.shape
    k = centroids.shape[0]
    return pl.pallas_call(
        _vq_argmin_kernel,
        in_specs=[
            pl.BlockSpec(memory_space=pltpu.MemorySpace.HBM),
            pl.BlockSpec(memory_space=pltpu.MemorySpace.HBM),
        ],
        out_shape=jax.ShapeDtypeStruct((n,), jnp.int32),
        scratch_shapes=[
            pltpu.VMEM((n, d), jnp.float32),
            pltpu.VMEM((k, d), jnp.float32),
            pltpu.SemaphoreType.DMA,
            pltpu.SemaphoreType.DMA,
        ],
    )(x, centroids)


# speedup vs baseline: 1.0373x; 1.0373x over previous
import jax
import jax.numpy as jnp
from jax.experimental import pallas as pl

_LB = 128


def _vq_argmin_kernel(x_ref, c_ref, out_ref):
    ct2 = c_ref[...].T * -2.0
    k = ct2.shape[1]
    cnorm = 0.25 * jnp.sum(ct2 * ct2, axis=0, keepdims=True)
    g2 = jnp.dot(x_ref[...], ct2, preferred_element_type=jnp.float32,
                 precision=jax.lax.Precision.HIGHEST)

    m8 = cnorm[:, :_LB] + g2[:, :_LB]
    a8 = jnp.zeros(m8.shape, jnp.int32)
    for c in range(1, k // _LB):
        s_c = cnorm[:, c * _LB:(c + 1) * _LB] + g2[:, c * _LB:(c + 1) * _LB]
        lt = s_c < m8
        m8 = jnp.where(lt, s_c, m8)
        a8 = jnp.where(lt, c, a8)
    lane = jax.lax.broadcasted_iota(jnp.int32, m8.shape, 1)
    idxp = a8 * _LB + lane
    m_t = m8.T
    i_t = idxp.T
    m = jnp.min(m_t, axis=0, keepdims=True)
    idx = jnp.min(jnp.where(m_t == m, i_t, k), axis=0)
    out_ref[...] = idx


def kernel(x, centroids):
    n, d = x.shape
    return pl.pallas_call(
        _vq_argmin_kernel,
        out_shape=jax.ShapeDtypeStruct((n,), jnp.int32),
    )(x, centroids)
